# baseline (speedup 1.0000x reference)
import functools
import jax
import jax.numpy as jnp
from jax import lax
from jax.experimental import pallas as pl
from jax.experimental.pallas import tpu as pltpu
from jax.experimental.pallas import tpu_sc as plsc

B = 4096
D = 768
NUM_TASKS = 8
HEAD_DIM = 64
PADW = 128
NPAIR = NUM_TASKS // 2
BLK = 512


def _heads_kernel(x_ref, w_ref, b_ref, h_ref, t_ref, w2d_ref):
    @pl.when(pl.program_id(0) == 0)
    def _build_w2d():
        for tt in range(NUM_TASKS):
            w2d_ref[:, tt * HEAD_DIM:(tt + 1) * HEAD_DIM] = w_ref[tt]

    xb = x_ref[...]
    feats = xb[:, :D]
    task = xb[:, D:]
    t = jnp.argmax(task, axis=-1)
    fo = jnp.tanh(feats)
    H = jnp.dot(fo, w2d_ref[...], preferred_element_type=jnp.float32)
    H = H + b_ref[...]
    for j in range(NPAIR):
        swap = ((t >> 1) == j) & ((t & 1) == 1)
        sel = swap[:, None]
        lo = H[:, j * PADW:j * PADW + HEAD_DIM]
        hi = H[:, j * PADW + HEAD_DIM:(j + 1) * PADW]
        h_ref[:, j * PADW:j * PADW + HEAD_DIM] = jnp.where(sel, hi, lo)
        h_ref[:, j * PADW + HEAD_DIM:(j + 1) * PADW] = jnp.where(sel, lo, hi)
    t_ref[0, 0, :] = t.reshape(1, 1, BLK)[0, 0, :]


_SC_INFO = plsc.get_sparse_core_info()
_NC = _SC_INFO.num_cores
_NL = _SC_INFO.num_lanes
_NW = _NC * _SC_INFO.num_subcores
_BPW = B // _NW


def _sc_gather(h_hbm, t_hbm, out_hbm, t_v, idx_v, rows_v, sem):
    wid = lax.axis_index("s") * _NC + lax.axis_index("c")
    base = wid * _BPW
    pltpu.sync_copy(t_hbm.at[pl.ds(base, _BPW)], t_v)
    for j in range(_BPW // _NL):
        tok = jnp.full((_NL,), base + j * _NL, jnp.int32) + lax.iota(jnp.int32, _NL)
        tj = t_v[pl.ds(j * _NL, _NL)]
        idx_v[pl.ds(j * _NL, _NL)] = tok * NPAIR + (tj >> 1)
    pltpu.async_copy(h_hbm.at[idx_v], rows_v, sem).wait()
    pltpu.sync_copy(rows_v, out_hbm.at[pl.ds(base, _BPW)])


def kernel(x, W_gating, gating_bias, W_heads, b_heads):
    b2d = b_heads.reshape(1, NUM_TASKS * HEAD_DIM)
    grid = (B // BLK,)
    H, t3 = pl.pallas_call(
        _heads_kernel,
        grid=grid,
        in_specs=[
            pl.BlockSpec((BLK, D + NUM_TASKS), lambda i: (i, 0)),
            pl.BlockSpec((NUM_TASKS, D, HEAD_DIM), lambda i: (0, 0, 0)),
            pl.BlockSpec((1, NUM_TASKS * HEAD_DIM), lambda i: (0, 0)),
        ],
        out_specs=[
            pl.BlockSpec((BLK, NUM_TASKS * HEAD_DIM), lambda i: (i, 0)),
            pl.BlockSpec((1, 1, BLK), lambda i: (i, 0, 0)),
        ],
        out_shape=[
            jax.ShapeDtypeStruct((B, NUM_TASKS * HEAD_DIM), jnp.float32),
            jax.ShapeDtypeStruct((B // BLK, 1, BLK), jnp.int32),
        ],
        scratch_shapes=[pltpu.VMEM((D, NUM_TASKS * HEAD_DIM), jnp.float32)],
    )(x, W_heads, b2d)

    table = H.reshape(B * NPAIR, PADW)
    t = t3.reshape(B)

    mesh = plsc.VectorSubcoreMesh(core_axis_name="c", subcore_axis_name="s")
    sc = functools.partial(
        pl.kernel,
        mesh=mesh,
        out_type=jax.ShapeDtypeStruct((B, PADW), jnp.float32),
        scratch_types=[
            pltpu.VMEM((_BPW,), jnp.int32),
            pltpu.VMEM((_BPW,), jnp.int32),
            pltpu.VMEM((_BPW, PADW), jnp.float32),
            pltpu.SemaphoreType.DMA,
        ],
    )(_sc_gather)
    return sc(table, t)[:, :HEAD_DIM]


# speedup vs baseline: 1.1709x; 1.1709x over previous
import functools
import jax
import jax.numpy as jnp
from jax import lax
from jax.experimental import pallas as pl
from jax.experimental.pallas import tpu as pltpu
from jax.experimental.pallas import tpu_sc as plsc

B = 4096
D = 768
NUM_TASKS = 8
HEAD_DIM = 64
PADW = 128
BLK = 1024


def _heads_kernel(x_ref, w_ref, b_ref, h_ref, t_ref, w2d_ref):
    @pl.when(pl.program_id(0) == 0)
    def _build_w2d():
        for tt in range(NUM_TASKS):
            w2d_ref[:, tt * HEAD_DIM:(tt + 1) * HEAD_DIM] = w_ref[tt]

    xb = x_ref[...]
    feats = xb[:, :D]
    task = xb[:, D:]
    t = jnp.argmax(task, axis=-1)
    fo = jnp.tanh(feats)
    H = jnp.dot(fo, w2d_ref[...], preferred_element_type=jnp.float32)
    H = H + b_ref[...]
    h_ref[:, :, :HEAD_DIM] = H.reshape(BLK, NUM_TASKS, HEAD_DIM)
    t_ref[0, 0, :] = t.reshape(1, 1, BLK)[0, 0, :]


_SC_INFO = plsc.get_sparse_core_info()
_NC = _SC_INFO.num_cores
_NL = _SC_INFO.num_lanes
_NW = _NC * _SC_INFO.num_subcores
_BPW = B // _NW


def _sc_gather(h_hbm, t_hbm, out_hbm, t_v, idx_v, rows_v, sem):
    wid = lax.axis_index("s") * _NC + lax.axis_index("c")
    base = wid * _BPW
    pltpu.sync_copy(t_hbm.at[pl.ds(base, _BPW)], t_v)
    for j in range(_BPW // _NL):
        tok = jnp.full((_NL,), base + j * _NL, jnp.int32) + lax.iota(jnp.int32, _NL)
        idx_v[pl.ds(j * _NL, _NL)] = tok * NUM_TASKS + t_v[pl.ds(j * _NL, _NL)]
    pltpu.async_copy(h_hbm.at[idx_v], rows_v, sem).wait()
    pltpu.sync_copy(rows_v, out_hbm.at[pl.ds(base, _BPW)])


def kernel(x, W_gating, gating_bias, W_heads, b_heads):
    b2d = b_heads.reshape(1, NUM_TASKS * HEAD_DIM)
    grid = (B // BLK,)
    H4, t3 = pl.pallas_call(
        _heads_kernel,
        grid=grid,
        in_specs=[
            pl.BlockSpec((BLK, D + NUM_TASKS), lambda i: (i, 0)),
            pl.BlockSpec((NUM_TASKS, D, HEAD_DIM), lambda i: (0, 0, 0)),
            pl.BlockSpec((1, NUM_TASKS * HEAD_DIM), lambda i: (0, 0)),
        ],
        out_specs=[
            pl.BlockSpec((BLK, NUM_TASKS, PADW), lambda i: (i, 0, 0)),
            pl.BlockSpec((1, 1, BLK), lambda i: (i, 0, 0)),
        ],
        out_shape=[
            jax.ShapeDtypeStruct((B, NUM_TASKS, PADW), jnp.float32),
            jax.ShapeDtypeStruct((B // BLK, 1, BLK), jnp.int32),
        ],
        scratch_shapes=[pltpu.VMEM((D, NUM_TASKS * HEAD_DIM), jnp.float32)],
    )(x, W_heads, b2d)

    table = H4.reshape(B * NUM_TASKS, PADW)
    t = t3.reshape(B)

    mesh = plsc.VectorSubcoreMesh(core_axis_name="c", subcore_axis_name="s")
    sc = functools.partial(
        pl.kernel,
        mesh=mesh,
        out_type=jax.ShapeDtypeStruct((B, PADW), jnp.float32),
        scratch_types=[
            pltpu.VMEM((_BPW,), jnp.int32),
            pltpu.VMEM((_BPW,), jnp.int32),
            pltpu.VMEM((_BPW, PADW), jnp.float32),
            pltpu.SemaphoreType.DMA,
        ],
    )(_sc_gather)
    return sc(table, t)[:, :HEAD_DIM]
